# Initial kernel scaffold; baseline (speedup 1.0000x reference)
import jax
import jax.numpy as jnp
from jax.experimental import pallas as pl


def kernel(pred_R, pred_t, pts_model, pts_gt, model_index, device):
    raise NotImplementedError("write your pallas kernel here")


# speedup vs baseline: 1.0586x; 1.0586x over previous
import jax
import jax.numpy as jnp
from jax.experimental import pallas as pl

_N = 8192
_QBLK = 1024
_CH = 2048
_LANES = 128


def _loss_body(p_ref, gt_t_ref, g_ref, r_ref, t_ref, sym_ref, asym_ref):
    step = pl.program_id(0)
    p = p_ref[...]
    r = r_ref[...]
    q = jnp.dot(p, r.T, preferred_element_type=jnp.float32) + t_ref[...]

    diff = q - g_ref[...]
    d_asym = jnp.sum(diff * diff, axis=1, keepdims=True)
    asym_part = jnp.sum(jnp.sqrt(d_asym), keepdims=True)

    qq = jnp.sum(q * q, axis=1, keepdims=True)
    qh0 = -2.0 * q[:, 0:1]
    qh1 = -2.0 * q[:, 1:2]
    qh2 = -2.0 * q[:, 2:3]
    qb = q.astype(jnp.bfloat16).astype(jnp.float32)
    qb0 = -2.0 * qb[:, 0:1]
    qb1 = -2.0 * qb[:, 1:2]
    qb2 = -2.0 * qb[:, 2:3]
    gt = gt_t_ref[...]
    gb = gt.astype(jnp.bfloat16).astype(jnp.float32)
    gg = gt[0:1, :] * gt[0:1, :] + gt[1:2, :] * gt[1:2, :] + gt[2:3, :] * gt[2:3, :]

    inf = jnp.float32(jnp.inf)
    smin = jnp.full((_QBLK, _LANES), inf, dtype=jnp.float32)
    emin = jnp.full((_QBLK, _LANES), inf, dtype=jnp.float32)
    for c in range(_N // _CH):
        lo, hi = c * _CH, (c + 1) * _CH
        base = gg[:, lo:hi]
        s = (base
             + qb0 * gb[0:1, lo:hi]
             + qb1 * gb[1:2, lo:hi]
             + qb2 * gb[2:3, lo:hi])
        e = (base
             + qh0 * gt[0:1, lo:hi]
             + qh1 * gt[1:2, lo:hi]
             + qh2 * gt[2:3, lo:hi])
        for k in range(_CH // _LANES):
            ss = s[:, k * _LANES:(k + 1) * _LANES]
            ee = e[:, k * _LANES:(k + 1) * _LANES]
            upd = ss < smin
            smin = jnp.where(upd, ss, smin)
            emin = jnp.where(upd, ee, emin)
    srow = jnp.min(smin, axis=1, keepdims=True)
    erow = jnp.min(jnp.where(smin == srow, emin, inf), axis=1, keepdims=True)
    dmin = qq + erow
    sym_part = jnp.sum(jnp.sqrt(jnp.maximum(dmin, 0.0)), keepdims=True)

    @pl.when(step == 0)
    def _init():
        sym_ref[...] = jnp.zeros((1, 1), jnp.float32)
        asym_ref[...] = jnp.zeros((1, 1), jnp.float32)

    sym_ref[...] += sym_part
    asym_ref[...] += asym_part


def kernel(pred_R, pred_t, pts_model, pts_gt, model_index, device):
    P = pts_model[0]
    G = pts_gt[0]
    GT = G.T
    R = pred_R[0]
    T = pred_t

    sym_sum, asym_sum = pl.pallas_call(
        _loss_body,
        grid=(_N // _QBLK,),
        in_specs=[
            pl.BlockSpec((_QBLK, 3), lambda i: (i, 0)),
            pl.BlockSpec((3, _N), lambda i: (0, 0)),
            pl.BlockSpec((_QBLK, 3), lambda i: (i, 0)),
            pl.BlockSpec((3, 3), lambda i: (0, 0)),
            pl.BlockSpec((1, 3), lambda i: (0, 0)),
        ],
        out_specs=[
            pl.BlockSpec((1, 1), lambda i: (0, 0)),
            pl.BlockSpec((1, 1), lambda i: (0, 0)),
        ],
        out_shape=[
            jax.ShapeDtypeStruct((1, 1), jnp.float32),
            jax.ShapeDtypeStruct((1, 1), jnp.float32),
        ],
    )(P, GT, G, R, T)

    is_sym = model_index.reshape(-1)[0] == 0
    loss = jnp.where(is_sym, sym_sum[0, 0], asym_sum[0, 0]) / _N
    return loss.reshape(1)


# speedup vs baseline: 1.6198x; 1.5302x over previous
import jax
import jax.numpy as jnp
from jax.experimental import pallas as pl

_N = 8192
_QBLK = 1024
_CH = 2048
_LANES = 128


def _loss_body(p_ref, gt_t_ref, g_ref, r_ref, t_ref, sym_ref, asym_ref):
    step = pl.program_id(0)
    p = p_ref[...]
    r = r_ref[...]
    q = jnp.dot(p, r.T, preferred_element_type=jnp.float32) + t_ref[...]

    diff = q - g_ref[...]
    d_asym = jnp.sum(diff * diff, axis=1, keepdims=True)
    asym_part = jnp.sum(jnp.sqrt(d_asym), keepdims=True)

    qq = jnp.sum(q * q, axis=1, keepdims=True)
    qb = q.astype(jnp.bfloat16)
    cq = q - qb.astype(jnp.float32)
    cqb = cq.astype(jnp.bfloat16)
    lhs1 = -2.0 * qb
    lhs2 = jnp.concatenate([lhs1, -2.0 * cqb], axis=1)

    gt = gt_t_ref[...]
    gb = gt.astype(jnp.bfloat16)
    cg = gt - gb.astype(jnp.float32)
    cgb = cg.astype(jnp.bfloat16)
    rhs2 = jnp.concatenate([cgb, gb], axis=0)
    gg = gt[0:1, :] * gt[0:1, :] + gt[1:2, :] * gt[1:2, :] + gt[2:3, :] * gt[2:3, :]

    dn = (((1,), (0,)), ((), ()))
    inf = jnp.float32(jnp.inf)
    smin = jnp.full((_QBLK, _LANES), inf, dtype=jnp.float32)
    emin = jnp.full((_QBLK, _LANES), inf, dtype=jnp.float32)
    for c in range(_N // _CH):
        lo, hi = c * _CH, (c + 1) * _CH
        m1 = jax.lax.dot_general(lhs1, gb[:, lo:hi], dn,
                                 preferred_element_type=jnp.float32)
        m23 = jax.lax.dot_general(lhs2, rhs2[:, lo:hi], dn,
                                  preferred_element_type=jnp.float32)
        s = gg[:, lo:hi] + m1
        e = s + m23
        for k in range(_CH // _LANES):
            ss = s[:, k * _LANES:(k + 1) * _LANES]
            ee = e[:, k * _LANES:(k + 1) * _LANES]
            upd = ss < smin
            smin = jnp.where(upd, ss, smin)
            emin = jnp.where(upd, ee, emin)
    srow = jnp.min(smin, axis=1, keepdims=True)
    erow = jnp.min(jnp.where(smin == srow, emin, inf), axis=1, keepdims=True)
    dmin = qq + erow
    sym_part = jnp.sum(jnp.sqrt(jnp.maximum(dmin, 0.0)), keepdims=True)

    @pl.when(step == 0)
    def _init():
        sym_ref[...] = jnp.zeros((1, 1), jnp.float32)
        asym_ref[...] = jnp.zeros((1, 1), jnp.float32)

    sym_ref[...] += sym_part
    asym_ref[...] += asym_part


def kernel(pred_R, pred_t, pts_model, pts_gt, model_index, device):
    P = pts_model[0]
    G = pts_gt[0]
    GT = G.T
    R = pred_R[0]
    T = pred_t

    sym_sum, asym_sum = pl.pallas_call(
        _loss_body,
        grid=(_N // _QBLK,),
        in_specs=[
            pl.BlockSpec((_QBLK, 3), lambda i: (i, 0)),
            pl.BlockSpec((3, _N), lambda i: (0, 0)),
            pl.BlockSpec((_QBLK, 3), lambda i: (i, 0)),
            pl.BlockSpec((3, 3), lambda i: (0, 0)),
            pl.BlockSpec((1, 3), lambda i: (0, 0)),
        ],
        out_specs=[
            pl.BlockSpec((1, 1), lambda i: (0, 0)),
            pl.BlockSpec((1, 1), lambda i: (0, 0)),
        ],
        out_shape=[
            jax.ShapeDtypeStruct((1, 1), jnp.float32),
            jax.ShapeDtypeStruct((1, 1), jnp.float32),
        ],
    )(P, GT, G, R, T)

    is_sym = model_index.reshape(-1)[0] == 0
    loss = jnp.where(is_sym, sym_sum[0, 0], asym_sum[0, 0]) / _N
    return loss.reshape(1)
